# baseline (speedup 1.0000x reference)
import functools

import jax
import jax.numpy as jnp
from jax import lax
from jax.experimental import pallas as pl
from jax.experimental.pallas import tpu as pltpu
from jax.experimental.pallas import tpu_sc as plsc

VOCAB = 1000000
D = 64
B = 4096
L = 200
N = B * L

NC = 2
NS = 16
NW = NC * NS
PER_W = N // NW
IDX_MINOR = 128
IDX_ROWS = PER_W // IDX_MINOR
CHUNK = 512
ROWS_PER_CHUNK = CHUNK // IDX_MINOR
NCHUNK = PER_W // CHUNK
SCALE = 8.0


@functools.partial(
    pl.kernel,
    out_type=jax.ShapeDtypeStruct((N, D), jnp.float32),
    mesh=plsc.VectorSubcoreMesh(core_axis_name="c", subcore_axis_name="s"),
    scratch_types=[
        pltpu.VMEM((IDX_ROWS, IDX_MINOR), jnp.int32),
        pltpu.VMEM((CHUNK, D), jnp.float32),
        pltpu.SemaphoreType.DMA,
        pltpu.SemaphoreType.DMA,
    ],
    compiler_params=pltpu.CompilerParams(use_tc_tiling_on_sc=False),
)
def _embed(x_hbm, table_hbm, out_hbm, idx_v, rows_v, gsem, osem):
    wid = lax.axis_index("s") * NC + lax.axis_index("c")
    pltpu.sync_copy(x_hbm.at[wid], idx_v)
    out_base = wid * PER_W

    def chunk_body(c, carry):
        cps = [
            pltpu.async_copy(
                table_hbm.at[idx_v.at[c * ROWS_PER_CHUNK + j]],
                rows_v.at[pl.ds(j * IDX_MINOR, IDX_MINOR)],
                gsem,
            )
            for j in range(ROWS_PER_CHUNK)
        ]
        for cp in cps:
            cp.wait()

        def scale_row(i, carry2):
            for j in range(D // 16):
                rows_v[i, pl.ds(j * 16, 16)] = rows_v[i, pl.ds(j * 16, 16)] * SCALE
            return carry2

        lax.fori_loop(0, CHUNK, scale_row, 0)
        pltpu.async_copy(
            rows_v, out_hbm.at[pl.ds(out_base + c * CHUNK, CHUNK)], osem
        ).wait()
        return carry

    lax.fori_loop(0, NCHUNK, chunk_body, 0)


def kernel(x, table):
    xf = x.astype(jnp.int32).reshape(NW, IDX_ROWS, IDX_MINOR)
    out = _embed(xf, table)
    return out.reshape(B, L, D)


# speedup vs baseline: 1.0828x; 1.0828x over previous
import functools

import jax
import jax.numpy as jnp
from jax import lax
from jax.experimental import pallas as pl
from jax.experimental.pallas import tpu as pltpu
from jax.experimental.pallas import tpu_sc as plsc

D = 64
B = 4096
L = 200
N = B * L

NC = 2
NS = 16
NW = NC * NS
PER_W = N // NW
IDX_MINOR = 128
IDX_ROWS = PER_W // IDX_MINOR
CHUNK = 256
GATHERS_PER_CHUNK = CHUNK // IDX_MINOR
NCHUNK = PER_W // CHUNK
NBUF = 4
NOUTER = NCHUNK // NBUF
SCALE = 8.0


@functools.partial(
    pl.kernel,
    out_type=jax.ShapeDtypeStruct((N, D), jnp.float32),
    mesh=plsc.VectorSubcoreMesh(core_axis_name="c", subcore_axis_name="s"),
    scratch_types=[
        pltpu.VMEM((IDX_ROWS, IDX_MINOR), jnp.int32),
        [pltpu.VMEM((CHUNK, D), jnp.float32) for _ in range(NBUF)],
        [pltpu.SemaphoreType.DMA for _ in range(NBUF)],
        [pltpu.SemaphoreType.DMA for _ in range(NBUF)],
    ],
    compiler_params=pltpu.CompilerParams(use_tc_tiling_on_sc=False),
)
def _embed(x_hbm, table_hbm, out_hbm, idx_v, rows, gsems, osems):
    wid = lax.axis_index("s") * NC + lax.axis_index("c")
    pltpu.sync_copy(x_hbm.at[wid], idx_v)
    out_base = wid * PER_W

    def start_gather(c, buf, sem):
        for j in range(GATHERS_PER_CHUNK):
            pltpu.async_copy(
                table_hbm.at[idx_v.at[c * GATHERS_PER_CHUNK + j]],
                buf.at[pl.ds(j * IDX_MINOR, IDX_MINOR)],
                sem,
            )

    def wait_gather(c, buf, sem):
        for j in range(GATHERS_PER_CHUNK):
            pltpu.make_async_copy(
                table_hbm.at[idx_v.at[c * GATHERS_PER_CHUNK + j]],
                buf.at[pl.ds(j * IDX_MINOR, IDX_MINOR)],
                sem,
            ).wait()

    def start_out(c, buf, sem):
        pltpu.async_copy(buf, out_hbm.at[pl.ds(out_base + c * CHUNK, CHUNK)], sem)

    def wait_out(buf, sem):
        pltpu.make_async_copy(buf, out_hbm.at[pl.ds(0, CHUNK)], sem).wait()

    def scale(buf):
        def scale_row(i, carry):
            for j in range(D // 16):
                buf[i, pl.ds(j * 16, 16)] = buf[i, pl.ds(j * 16, 16)] * SCALE
            return carry

        lax.fori_loop(0, CHUNK, scale_row, 0)

    for b in range(NBUF - 1):
        start_gather(b, rows[b], gsems[b])

    def outer(p, carry):
        for b in range(NBUF):
            g = p * NBUF + b
            nb = (b + NBUF - 1) % NBUF
            nxt = g + NBUF - 1

            @pl.when(nxt < NCHUNK)
            def _issue():
                if b == 0:

                    @pl.when(p > 0)
                    def _():
                        wait_out(rows[nb], osems[nb])

                else:
                    wait_out(rows[nb], osems[nb])
                start_gather(nxt, rows[nb], gsems[nb])

            wait_gather(g, rows[b], gsems[b])
            scale(rows[b])
            start_out(g, rows[b], osems[b])
        return carry

    lax.fori_loop(0, NOUTER, outer, 0)

    for b in range(NBUF):
        wait_out(rows[b], osems[b])


def kernel(x, table):
    xf = x.astype(jnp.int32).reshape(NW, IDX_ROWS, IDX_MINOR)
    out = _embed(xf, table)
    return out.reshape(B, L, D)
